# baseline (speedup 1.0000x reference)
import functools

import jax
import jax.numpy as jnp
from jax import lax
from jax.experimental import pallas as pl
from jax.experimental.pallas import tpu as pltpu
from jax.experimental.pallas import tpu_sc as plsc

B = 16384
PK = 100001
D = 64
TV = 1000
T = 16

NC = 2
NS = 16
NW = NC * NS
BPW = B // NW
PK_CH = 256


def _sc_gather(item_id, title, pk_table, text_flat):
    mesh = plsc.VectorSubcoreMesh(core_axis_name="c", subcore_axis_name="s",
                                  num_cores=NC, num_subcores=NS)

    @functools.partial(
        pl.kernel,
        out_type=[
            jax.ShapeDtypeStruct((B, D), jnp.float32),
            jax.ShapeDtypeStruct((B, D), jnp.float32),
        ],
        mesh=mesh,
        compiler_params=pltpu.CompilerParams(needs_layout_passes=False,
                                             use_tc_tiling_on_sc=False),
        scratch_types=[
            pltpu.VMEM((BPW,), jnp.int32),
            pltpu.VMEM((BPW, T), jnp.int32),
            pltpu.VMEM((TV * D,), jnp.float32),
            pltpu.VMEM((PK_CH, D), jnp.float32),
            pltpu.VMEM((BPW, D), jnp.float32),
            pltpu.SemaphoreType.DMA,
        ],
    )
    def k(item_hbm, title_hbm, pk_hbm, text_hbm, pkrows_hbm, tsum_hbm,
          idx_v, title_v, table_v, rows_v, tsum_v, sem):
        wid = lax.axis_index("s") * NC + lax.axis_index("c")
        base = wid * BPW

        pltpu.sync_copy(item_hbm.at[pl.ds(base, BPW)], idx_v)
        pltpu.sync_copy(text_hbm, table_v)
        pltpu.sync_copy(title_hbm.at[pl.ds(base, BPW)], title_v)

        zero16 = jnp.zeros((16,), jnp.float32)
        for kk in range(D // 16):
            table_v[pl.ds(kk * 16, 16)] = zero16

        iotas = [lax.broadcasted_iota(jnp.int32, (16,), 0) + kk * 16
                 for kk in range(D // 16)]

        def pool_rows(lo):
            @plsc.parallel_loop(lo, lo + PK_CH, unroll=4)
            def _(r):
                accs = [jnp.zeros((16,), jnp.float32)
                        for _ in range(D // 16)]
                trow = title_v[r, :] * D
                for t in range(T):
                    off = trow[t]
                    for kk in range(D // 16):
                        g = plsc.load_gather(table_v, [off + iotas[kk]])
                        accs[kk] = accs[kk] + g
                for kk in range(D // 16):
                    tsum_v[r, pl.ds(kk * 16, 16)] = accs[kk]

        for c in range(BPW // PK_CH):
            cp = pltpu.async_copy(
                pk_hbm.at[idx_v.at[pl.ds(c * PK_CH, PK_CH)]], rows_v, sem)
            pool_rows(c * PK_CH)
            cp.wait()
            pltpu.sync_copy(rows_v,
                            pkrows_hbm.at[pl.ds(base + c * PK_CH, PK_CH)])
        pltpu.sync_copy(tsum_v, tsum_hbm.at[pl.ds(base, BPW)])

    return k(item_id, title, pk_table, text_flat)


def _tc_fnn(pk_rows, tsum, title, cat, brand, pt,
            W1pk, W1cat, W1brand, W1text, W1pt, b1, W2, b2):
    BLK = 1024
    grid = (B // BLK,)
    row_spec2 = lambda w: pl.BlockSpec((BLK, w), lambda i: (i, 0))
    full_spec = lambda a, b: pl.BlockSpec((a, b), lambda i: (0, 0))

    def body(pk_ref, tsum_ref, title_ref, cat_ref, brand_ref, pt_ref,
             w1pk_ref, w1c_ref, w1b_ref, w1t_ref, w1pt_ref, b1_ref,
             w2_ref, b2_ref, out_ref):
        ttl = title_ref[...]
        cnt = jnp.sum((ttl != 0).astype(jnp.float32), axis=1, keepdims=True)
        text = tsum_ref[...] / jnp.maximum(cnt, 1.0)
        oh_c = (cat_ref[...] ==
                lax.broadcasted_iota(jnp.int32, (BLK, 16), 1)).astype(
                    jnp.float32)
        oh_b = (brand_ref[...] ==
                lax.broadcasted_iota(jnp.int32, (BLK, 8), 1)).astype(
                    jnp.float32)
        x1 = jnp.dot(pk_ref[...], w1pk_ref[...],
                     preferred_element_type=jnp.float32)
        x1 += jnp.dot(text, w1t_ref[...],
                      preferred_element_type=jnp.float32)
        x1 += jnp.dot(oh_c, w1c_ref[...],
                      preferred_element_type=jnp.float32)
        x1 += jnp.dot(oh_b, w1b_ref[...],
                      preferred_element_type=jnp.float32)
        x1 += jnp.dot(pt_ref[...], w1pt_ref[...],
                      preferred_element_type=jnp.float32)
        h = jnp.maximum(x1 + b1_ref[...], 0.0)
        out_ref[...] = jnp.dot(h, w2_ref[...],
                               preferred_element_type=jnp.float32) + b2_ref[...]

    return pl.pallas_call(
        body,
        grid=grid,
        in_specs=[
            row_spec2(D),
            row_spec2(D),
            row_spec2(T),
            row_spec2(1),
            row_spec2(1),
            row_spec2(2),
            full_spec(D, D),
            full_spec(16, D),
            full_spec(8, D),
            full_spec(D, D),
            full_spec(2, D),
            full_spec(1, D),
            full_spec(D, D),
            full_spec(1, D),
        ],
        out_specs=row_spec2(D),
        out_shape=jax.ShapeDtypeStruct((B, D), jnp.float32),
    )(pk_rows, tsum, title, cat, brand, pt,
      W1pk, W1cat, W1brand, W1text, W1pt, b1, W2, b2)


def kernel(item_id, category, brand, title, price, created_at,
           pk_table, text_table, W1, b1, W2, b2):
    item_id = item_id.astype(jnp.int32)
    title_i = title.astype(jnp.int32)
    pk_rows, tsum = _sc_gather(item_id, title_i, pk_table,
                               text_table.reshape(-1))
    pt = jnp.stack([price, created_at], axis=1)
    out = _tc_fnn(
        pk_rows, tsum, title_i,
        category.astype(jnp.int32).reshape(B, 1),
        brand.astype(jnp.int32).reshape(B, 1),
        pt,
        W1[0:D],
        W1[D:D + 16],
        W1[D + 16:D + 24],
        W1[D + 24:D + 24 + D],
        W1[D + 24 + D:],
        b1.reshape(1, D),
        W2,
        b2.reshape(1, D),
    )
    return out


# speedup vs baseline: 1.3311x; 1.3311x over previous
import functools

import jax
import jax.numpy as jnp
from jax import lax
from jax.experimental import pallas as pl
from jax.experimental.pallas import tpu as pltpu
from jax.experimental.pallas import tpu_sc as plsc

B = 16384
PK = 100001
D = 64
TV = 1000
T = 16

NC = 2
NS = 16
NW = NC * NS
BPW = B // NW
PK_CH = 256


def _sc_gather(item_id, title, pk_table, text_flat):
    mesh = plsc.VectorSubcoreMesh(core_axis_name="c", subcore_axis_name="s",
                                  num_cores=NC, num_subcores=NS)

    @functools.partial(
        pl.kernel,
        out_type=[
            jax.ShapeDtypeStruct((B, D), jnp.float32),
            jax.ShapeDtypeStruct((B, D), jnp.float32),
        ],
        mesh=mesh,
        compiler_params=pltpu.CompilerParams(needs_layout_passes=False,
                                             use_tc_tiling_on_sc=False),
        scratch_types=[
            pltpu.VMEM((BPW,), jnp.int32),
            pltpu.VMEM((BPW, T), jnp.int32),
            pltpu.VMEM((TV * D,), jnp.float32),
            pltpu.VMEM((PK_CH, D), jnp.float32),
            pltpu.VMEM((BPW, D), jnp.float32),
            pltpu.SemaphoreType.DMA,
        ],
    )
    def k(item_hbm, title_hbm, pk_hbm, text_hbm, pkrows_hbm, tsum_hbm,
          idx_v, title_v, table_v, rows_v, tsum_v, sem):
        wid = lax.axis_index("s") * NC + lax.axis_index("c")
        base = wid * BPW

        pltpu.sync_copy(item_hbm.at[pl.ds(base, BPW)], idx_v)
        pltpu.sync_copy(text_hbm, table_v)
        pltpu.sync_copy(title_hbm.at[pl.ds(base, BPW)], title_v)

        zero16 = jnp.zeros((16,), jnp.float32)
        for kk in range(D // 16):
            table_v[pl.ds(kk * 16, 16)] = zero16

        iotas = [lax.broadcasted_iota(jnp.int32, (16,), 0) + kk * 16
                 for kk in range(D // 16)]

        def pool_rows(lo):
            def _(r, carry):
                accs = [jnp.zeros((16,), jnp.float32)
                        for _ in range(D // 16)]
                trow = title_v[r, :] * D
                for t in range(T):
                    off = trow[t]
                    for kk in range(D // 16):
                        g = plsc.load_gather(table_v, [off + iotas[kk]])
                        accs[kk] = accs[kk] + g
                for kk in range(D // 16):
                    tsum_v[r, pl.ds(kk * 16, 16)] = accs[kk]
                return carry
            lax.fori_loop(lo, lo + PK_CH, _, 0)

        for c in range(BPW // PK_CH):
            cp = pltpu.async_copy(
                pk_hbm.at[idx_v.at[pl.ds(c * PK_CH, PK_CH)]], rows_v, sem)
            pool_rows(c * PK_CH)
            cp.wait()
            pltpu.sync_copy(rows_v,
                            pkrows_hbm.at[pl.ds(base + c * PK_CH, PK_CH)])
        pltpu.sync_copy(tsum_v, tsum_hbm.at[pl.ds(base, BPW)])

    return k(item_id, title, pk_table, text_flat)


def _tc_fnn(pk_rows, tsum, title, cat, brand, pt,
            W1pk, W1cat, W1brand, W1text, W1pt, b1, W2, b2):
    BLK = 1024
    grid = (B // BLK,)
    row_spec2 = lambda w: pl.BlockSpec((BLK, w), lambda i: (i, 0))
    full_spec = lambda a, b: pl.BlockSpec((a, b), lambda i: (0, 0))

    def body(pk_ref, tsum_ref, title_ref, cat_ref, brand_ref, pt_ref,
             w1pk_ref, w1c_ref, w1b_ref, w1t_ref, w1pt_ref, b1_ref,
             w2_ref, b2_ref, out_ref):
        ttl = title_ref[...]
        cnt = jnp.sum((ttl != 0).astype(jnp.float32), axis=1, keepdims=True)
        text = tsum_ref[...] / jnp.maximum(cnt, 1.0)
        oh_c = (cat_ref[...] ==
                lax.broadcasted_iota(jnp.int32, (BLK, 16), 1)).astype(
                    jnp.float32)
        oh_b = (brand_ref[...] ==
                lax.broadcasted_iota(jnp.int32, (BLK, 8), 1)).astype(
                    jnp.float32)
        x1 = jnp.dot(pk_ref[...], w1pk_ref[...],
                     preferred_element_type=jnp.float32)
        x1 += jnp.dot(text, w1t_ref[...],
                      preferred_element_type=jnp.float32)
        x1 += jnp.dot(oh_c, w1c_ref[...],
                      preferred_element_type=jnp.float32)
        x1 += jnp.dot(oh_b, w1b_ref[...],
                      preferred_element_type=jnp.float32)
        x1 += jnp.dot(pt_ref[...], w1pt_ref[...],
                      preferred_element_type=jnp.float32)
        h = jnp.maximum(x1 + b1_ref[...], 0.0)
        out_ref[...] = jnp.dot(h, w2_ref[...],
                               preferred_element_type=jnp.float32) + b2_ref[...]

    return pl.pallas_call(
        body,
        grid=grid,
        in_specs=[
            row_spec2(D),
            row_spec2(D),
            row_spec2(T),
            row_spec2(1),
            row_spec2(1),
            row_spec2(2),
            full_spec(D, D),
            full_spec(16, D),
            full_spec(8, D),
            full_spec(D, D),
            full_spec(2, D),
            full_spec(1, D),
            full_spec(D, D),
            full_spec(1, D),
        ],
        out_specs=row_spec2(D),
        out_shape=jax.ShapeDtypeStruct((B, D), jnp.float32),
    )(pk_rows, tsum, title, cat, brand, pt,
      W1pk, W1cat, W1brand, W1text, W1pt, b1, W2, b2)


def kernel(item_id, category, brand, title, price, created_at,
           pk_table, text_table, W1, b1, W2, b2):
    item_id = item_id.astype(jnp.int32)
    title_i = title.astype(jnp.int32)
    pk_rows, tsum = _sc_gather(item_id, title_i, pk_table,
                               text_table.reshape(-1))
    pt = jnp.stack([price, created_at], axis=1)
    out = _tc_fnn(
        pk_rows, tsum, title_i,
        category.astype(jnp.int32).reshape(B, 1),
        brand.astype(jnp.int32).reshape(B, 1),
        pt,
        W1[0:D],
        W1[D:D + 16],
        W1[D + 16:D + 24],
        W1[D + 24:D + 24 + D],
        W1[D + 24 + D:],
        b1.reshape(1, D),
        W2,
        b2.reshape(1, D),
    )
    return out


# speedup vs baseline: 1.5920x; 1.1960x over previous
import functools

import jax
import jax.numpy as jnp
from jax import lax
from jax.experimental import pallas as pl
from jax.experimental.pallas import tpu as pltpu
from jax.experimental.pallas import tpu_sc as plsc

B = 16384
HB = B // 2
PK = 100001
D = 64
TV = 1000
T = 16

NC = 2
NS = 16
NW = NC * NS
BPW = B // NW
PK_CH = 256

_MESH = dict(core_axis_name="c", subcore_axis_name="s",
             num_cores=NC, num_subcores=NS)
_SC_PARAMS = pltpu.CompilerParams(needs_layout_passes=False,
                                  use_tc_tiling_on_sc=False)


def _sc_pool(title, text_flat):
    @functools.partial(
        pl.kernel,
        out_type=jax.ShapeDtypeStruct((B, D), jnp.float32),
        mesh=plsc.VectorSubcoreMesh(**_MESH),
        compiler_params=_SC_PARAMS,
        scratch_types=[
            pltpu.VMEM((BPW, T), jnp.int32),
            pltpu.VMEM((TV * D,), jnp.float32),
            pltpu.VMEM((BPW, D), jnp.float32),
        ],
    )
    def k(title_hbm, text_hbm, text2_hbm, title_v, table_v, tsum_v):
        wid = lax.axis_index("s") * NC + lax.axis_index("c")
        base = wid * BPW

        pltpu.sync_copy(text_hbm, table_v)
        pltpu.sync_copy(title_hbm.at[pl.ds(base, BPW)], title_v)

        zero16 = jnp.zeros((16,), jnp.float32)
        for kk in range(D // 16):
            table_v[pl.ds(kk * 16, 16)] = zero16

        one16 = jnp.full((16,), 1.0, jnp.float32)

        def row_body(r, carry):
            accs = [jnp.zeros((16,), jnp.float32) for _ in range(D // 16)]
            trow = title_v[r, :]
            cnt = plsc.all_reduce_population_count(trow != 0)
            offs = trow * D
            for t in range(T):
                off = offs[t]
                for kk in range(D // 16):
                    g = table_v[pl.ds(off + kk * 16, 16)]
                    accs[kk] = accs[kk] + g
            scale = one16 / jnp.maximum(cnt.astype(jnp.float32), one16)
            for kk in range(D // 16):
                tsum_v[r, pl.ds(kk * 16, 16)] = accs[kk] * scale
            return carry

        lax.fori_loop(0, BPW, row_body, 0)
        pltpu.sync_copy(tsum_v, text2_hbm.at[pl.ds(base, BPW)])

    return k(title, text_flat)


def _sc_pk(item_id, pk_table, cat, brand, price, ts, wmisc_flat):
    @functools.partial(
        pl.kernel,
        out_type=[
            jax.ShapeDtypeStruct((B, D), jnp.float32),
            jax.ShapeDtypeStruct((B, D), jnp.float32),
        ],
        mesh=plsc.VectorSubcoreMesh(**_MESH),
        compiler_params=_SC_PARAMS,
        scratch_types=[
            pltpu.VMEM((BPW,), jnp.int32),
            pltpu.VMEM((BPW,), jnp.int32),
            pltpu.VMEM((BPW,), jnp.int32),
            pltpu.VMEM((BPW,), jnp.float32),
            pltpu.VMEM((BPW,), jnp.float32),
            pltpu.VMEM((27 * D,), jnp.float32),
            pltpu.VMEM((PK_CH, D), jnp.float32),
            pltpu.VMEM((BPW, D), jnp.float32),
            pltpu.SemaphoreType.DMA,
        ],
    )
    def k(item_hbm, pk_hbm, cat_hbm, brand_hbm, price_hbm, ts_hbm, wm_hbm,
          pk2_hbm, misc2_hbm,
          idx_v, cat_v, brand_v, price_v, ts_v, wm_v, rows_v, misc_v, sem):
        wid = lax.axis_index("s") * NC + lax.axis_index("c")
        base = wid * BPW
        pltpu.sync_copy(item_hbm.at[pl.ds(base, BPW)], idx_v)
        pltpu.sync_copy(cat_hbm.at[pl.ds(base, BPW)], cat_v)
        pltpu.sync_copy(brand_hbm.at[pl.ds(base, BPW)], brand_v)
        pltpu.sync_copy(price_hbm.at[pl.ds(base, BPW)], price_v)
        pltpu.sync_copy(ts_hbm.at[pl.ds(base, BPW)], ts_v)
        pltpu.sync_copy(wm_hbm, wm_v)

        wp = [wm_v[pl.ds(24 * D + kk * 16, 16)] for kk in range(D // 16)]
        wt = [wm_v[pl.ds(25 * D + kk * 16, 16)] for kk in range(D // 16)]
        b1r = [wm_v[pl.ds(26 * D + kk * 16, 16)] for kk in range(D // 16)]

        def misc_body(r16, carry):
            r = r16 * 16
            cvec = cat_v[pl.ds(r, 16)] * D
            bvec = (brand_v[pl.ds(r, 16)] + 16) * D
            pvec = price_v[pl.ds(r, 16)]
            tvec = ts_v[pl.ds(r, 16)]
            for j in range(16):
                co = cvec[j]
                bo = bvec[j]
                p = pvec[j]
                t = tvec[j]
                for kk in range(D // 16):
                    acc = (wm_v[pl.ds(co + kk * 16, 16)]
                           + wm_v[pl.ds(bo + kk * 16, 16)]
                           + p * wp[kk] + t * wt[kk] + b1r[kk])
                    misc_v[r + j, pl.ds(kk * 16, 16)] = acc
            return carry

        lax.fori_loop(0, BPW // 16, misc_body, 0)
        pltpu.sync_copy(misc_v, misc2_hbm.at[pl.ds(base, BPW)])

        for c in range(BPW // PK_CH):
            pltpu.async_copy(
                pk_hbm.at[idx_v.at[pl.ds(c * PK_CH, PK_CH)]], rows_v, sem
            ).wait()
            pltpu.sync_copy(rows_v,
                            pk2_hbm.at[pl.ds(base + c * PK_CH, PK_CH)])

    return k(item_id, pk_table, cat, brand, price, ts, wmisc_flat)


def _tc_fnn(pk2, text2, misc2, W1pk, W1text, W2, b2):
    BLK = 1024
    grid = (HB // BLK,)
    row_spec = pl.BlockSpec((BLK, 2 * D), lambda i: (i, 0))
    full_spec = pl.BlockSpec((D, D), lambda i: (0, 0))

    def body(pk_ref, text_ref, misc_ref, w1pk_ref, w1t_ref, w2_ref, b2_ref,
             out_ref):
        for h in (0, D):
            x1 = jnp.dot(pk_ref[:, h:h + D], w1pk_ref[...],
                         preferred_element_type=jnp.float32)
            x1 += jnp.dot(text_ref[:, h:h + D], w1t_ref[...],
                          preferred_element_type=jnp.float32)
            x1 += misc_ref[:, h:h + D]
            hrelu = jnp.maximum(x1, 0.0)
            out_ref[:, h:h + D] = jnp.dot(
                hrelu, w2_ref[...],
                preferred_element_type=jnp.float32) + b2_ref[...]

    return pl.pallas_call(
        body,
        grid=grid,
        in_specs=[
            row_spec,
            row_spec,
            row_spec,
            full_spec,
            full_spec,
            full_spec,
            pl.BlockSpec((1, D), lambda i: (0, 0)),
        ],
        out_specs=row_spec,
        out_shape=jax.ShapeDtypeStruct((HB, 2 * D), jnp.float32),
    )(pk2, text2, misc2, W1pk, W1text, W2, b2)


def kernel(item_id, category, brand, title, price, created_at,
           pk_table, text_table, W1, b1, W2, b2):
    item_id = item_id.astype(jnp.int32)
    title_i = title.astype(jnp.int32)
    text2 = _sc_pool(title_i, text_table.reshape(-1))
    wmisc = jnp.concatenate(
        [W1[D:D + 24], W1[D + 24 + D:], b1.reshape(1, D)], axis=0)
    pk_rows, misc_rows = _sc_pk(item_id, pk_table,
                        category.astype(jnp.int32), brand.astype(jnp.int32),
                        price, created_at, wmisc.reshape(-1))
    pk2 = pk_rows.reshape(HB, 2 * D)
    misc2 = misc_rows.reshape(HB, 2 * D)
    out2 = _tc_fnn(pk2, text2.reshape(HB, 2 * D), misc2,
                   W1[0:D], W1[D + 24:D + 24 + D], W2, b2.reshape(1, D))
    return out2.reshape(B, D)


# speedup vs baseline: 1.6573x; 1.0410x over previous
import functools

import jax
import jax.numpy as jnp
from jax import lax
from jax.experimental import pallas as pl
from jax.experimental.pallas import tpu as pltpu
from jax.experimental.pallas import tpu_sc as plsc

B = 16384
HB = B // 2
PK = 100001
D = 64
TV = 1000
T = 16

NC = 2
NS = 16
NW = NC * NS
BPW = B // NW
PK_CH = 256

_MESH = dict(core_axis_name="c", subcore_axis_name="s",
             num_cores=NC, num_subcores=NS)
_SC_PARAMS = pltpu.CompilerParams(needs_layout_passes=False,
                                  use_tc_tiling_on_sc=False)


def _sc_pool(title, text_flat):
    @functools.partial(
        pl.kernel,
        out_type=jax.ShapeDtypeStruct((B, D), jnp.float32),
        mesh=plsc.VectorSubcoreMesh(**_MESH),
        compiler_params=_SC_PARAMS,
        scratch_types=[
            pltpu.VMEM((BPW, T), jnp.int32),
            pltpu.VMEM((TV * D,), jnp.float32),
            pltpu.VMEM((BPW, D), jnp.float32),
        ],
    )
    def k(title_hbm, text_hbm, text2_hbm, title_v, table_v, tsum_v):
        wid = lax.axis_index("s") * NC + lax.axis_index("c")
        base = wid * BPW

        pltpu.sync_copy(text_hbm, table_v)
        pltpu.sync_copy(title_hbm.at[pl.ds(base, BPW)], title_v)

        zero16 = jnp.zeros((16,), jnp.float32)
        for kk in range(D // 16):
            table_v[pl.ds(kk * 16, 16)] = zero16

        one16 = jnp.full((16,), 1.0, jnp.float32)

        def row_body(r, carry):
            accs = [jnp.zeros((16,), jnp.float32) for _ in range(D // 16)]
            trow = title_v[r, :]
            cnt = plsc.all_reduce_population_count(trow != 0)
            offs = trow * D
            for t in range(T):
                off = offs[t]
                for kk in range(D // 16):
                    g = table_v[pl.ds(off + kk * 16, 16)]
                    accs[kk] = accs[kk] + g
            scale = one16 / jnp.maximum(cnt.astype(jnp.float32), one16)
            for kk in range(D // 16):
                tsum_v[r, pl.ds(kk * 16, 16)] = accs[kk] * scale
            return carry

        lax.fori_loop(0, BPW, row_body, 0)
        pltpu.sync_copy(tsum_v, text2_hbm.at[pl.ds(base, BPW)])

    return k(title, text_flat)


def _sc_pk(item_id, pk_table, combo, price, ts, wmisc_flat):
    @functools.partial(
        pl.kernel,
        out_type=[
            jax.ShapeDtypeStruct((B, D), jnp.float32),
            jax.ShapeDtypeStruct((B, D), jnp.float32),
        ],
        mesh=plsc.VectorSubcoreMesh(**_MESH),
        compiler_params=_SC_PARAMS,
        scratch_types=[
            pltpu.VMEM((BPW,), jnp.int32),
            pltpu.VMEM((BPW,), jnp.int32),
            pltpu.VMEM((BPW,), jnp.float32),
            pltpu.VMEM((BPW,), jnp.float32),
            pltpu.VMEM((130 * D,), jnp.float32),
            pltpu.VMEM((PK_CH, D), jnp.float32),
            pltpu.VMEM((BPW, D), jnp.float32),
            pltpu.SemaphoreType.DMA,
        ],
    )
    def k(item_hbm, pk_hbm, combo_hbm, price_hbm, ts_hbm, wm_hbm,
          pk2_hbm, misc2_hbm,
          idx_v, combo_v, price_v, ts_v, wm_v, rows_v, misc_v, sem):
        wid = lax.axis_index("s") * NC + lax.axis_index("c")
        base = wid * BPW
        pltpu.sync_copy(item_hbm.at[pl.ds(base, BPW)], idx_v)
        cp0 = pltpu.async_copy(
            pk_hbm.at[idx_v.at[pl.ds(0, PK_CH)]], rows_v, sem)
        pltpu.sync_copy(combo_hbm.at[pl.ds(base, BPW)], combo_v)
        pltpu.sync_copy(price_hbm.at[pl.ds(base, BPW)], price_v)
        pltpu.sync_copy(ts_hbm.at[pl.ds(base, BPW)], ts_v)
        pltpu.sync_copy(wm_hbm, wm_v)

        wp = [wm_v[pl.ds(128 * D + kk * 16, 16)] for kk in range(D // 16)]
        wt = [wm_v[pl.ds(129 * D + kk * 16, 16)] for kk in range(D // 16)]

        def misc_body(r16, carry):
            r = r16 * 16
            cvec = combo_v[pl.ds(r, 16)] * D
            pvec = price_v[pl.ds(r, 16)]
            tvec = ts_v[pl.ds(r, 16)]
            for j in range(16):
                co = cvec[j]
                p = pvec[j]
                t = tvec[j]
                for kk in range(D // 16):
                    acc = (wm_v[pl.ds(co + kk * 16, 16)]
                           + p * wp[kk] + t * wt[kk])
                    misc_v[r + j, pl.ds(kk * 16, 16)] = acc
            return carry

        lax.fori_loop(0, BPW // 16, misc_body, 0)
        pltpu.sync_copy(misc_v, misc2_hbm.at[pl.ds(base, BPW)])

        cp0.wait()
        pltpu.sync_copy(rows_v, pk2_hbm.at[pl.ds(base, PK_CH)])
        for c in range(1, BPW // PK_CH):
            pltpu.async_copy(
                pk_hbm.at[idx_v.at[pl.ds(c * PK_CH, PK_CH)]], rows_v, sem
            ).wait()
            pltpu.sync_copy(rows_v,
                            pk2_hbm.at[pl.ds(base + c * PK_CH, PK_CH)])

    return k(item_id, pk_table, combo, price, ts, wmisc_flat)


def _tc_fnn(pk2, text2, misc2, W1pk, W1text, W2, b2):
    BLK = 1024
    grid = (HB // BLK,)
    row_spec = pl.BlockSpec((BLK, 2 * D), lambda i: (i, 0))
    full_spec = pl.BlockSpec((D, D), lambda i: (0, 0))

    def body(pk_ref, text_ref, misc_ref, w1pk_ref, w1t_ref, w2_ref, b2_ref,
             out_ref):
        for h in (0, D):
            x1 = jnp.dot(pk_ref[:, h:h + D], w1pk_ref[...],
                         preferred_element_type=jnp.float32)
            x1 += jnp.dot(text_ref[:, h:h + D], w1t_ref[...],
                          preferred_element_type=jnp.float32)
            x1 += misc_ref[:, h:h + D]
            hrelu = jnp.maximum(x1, 0.0)
            out_ref[:, h:h + D] = jnp.dot(
                hrelu, w2_ref[...],
                preferred_element_type=jnp.float32) + b2_ref[...]

    return pl.pallas_call(
        body,
        grid=grid,
        in_specs=[
            row_spec,
            row_spec,
            row_spec,
            full_spec,
            full_spec,
            full_spec,
            pl.BlockSpec((1, D), lambda i: (0, 0)),
        ],
        out_specs=row_spec,
        out_shape=jax.ShapeDtypeStruct((HB, 2 * D), jnp.float32),
    )(pk2, text2, misc2, W1pk, W1text, W2, b2)


def kernel(item_id, category, brand, title, price, created_at,
           pk_table, text_table, W1, b1, W2, b2):
    item_id = item_id.astype(jnp.int32)
    title_i = title.astype(jnp.int32)
    text2 = _sc_pool(title_i, text_table.reshape(-1))
    combo_tab = (W1[D:D + 16][:, None, :] + W1[D + 16:D + 24][None, :, :]
                 + b1[None, None, :]).reshape(128, D)
    wmisc = jnp.concatenate([combo_tab, W1[D + 24 + D:]], axis=0)
    combo = category.astype(jnp.int32) * 8 + brand.astype(jnp.int32)
    pk_rows, misc_rows = _sc_pk(item_id, pk_table, combo,
                                price, created_at, wmisc.reshape(-1))
    pk2 = pk_rows.reshape(HB, 2 * D)
    misc2 = misc_rows.reshape(HB, 2 * D)
    out2 = _tc_fnn(pk2, text2.reshape(HB, 2 * D), misc2,
                   W1[0:D], W1[D + 24:D + 24 + D], W2, b2.reshape(1, D))
    return out2.reshape(B, D)


# speedup vs baseline: 1.7098x; 1.0317x over previous
import functools

import jax
import jax.numpy as jnp
from jax import lax
from jax.experimental import pallas as pl
from jax.experimental.pallas import tpu as pltpu
from jax.experimental.pallas import tpu_sc as plsc

B = 16384
HB = B // 2
PK = 100001
D = 64
TV = 1000
T = 16

NC = 2
NS = 16
NW = NC * NS
BPW = B // NW
PK_CH = 256

_MESH = dict(core_axis_name="c", subcore_axis_name="s",
             num_cores=NC, num_subcores=NS)
_SC_PARAMS = pltpu.CompilerParams(needs_layout_passes=False,
                                  use_tc_tiling_on_sc=False)


def _sc_pool(title, text_flat):
    @functools.partial(
        pl.kernel,
        out_type=jax.ShapeDtypeStruct((B, D), jnp.float32),
        mesh=plsc.VectorSubcoreMesh(**_MESH),
        compiler_params=_SC_PARAMS,
        scratch_types=[
            pltpu.VMEM((BPW, T), jnp.int32),
            pltpu.VMEM((TV * D,), jnp.float32),
            pltpu.VMEM((BPW, D), jnp.float32),
        ],
    )
    def k(title_hbm, text_hbm, text2_hbm, title_v, table_v, tsum_v):
        wid = lax.axis_index("s") * NC + lax.axis_index("c")
        base = wid * BPW

        pltpu.sync_copy(text_hbm, table_v)
        pltpu.sync_copy(title_hbm.at[pl.ds(base, BPW)], title_v)

        zero16 = jnp.zeros((16,), jnp.float32)
        for kk in range(D // 16):
            table_v[pl.ds(kk * 16, 16)] = zero16

        one16 = jnp.full((16,), 1.0, jnp.float32)

        def row_body(r, carry):
            accs = [jnp.zeros((16,), jnp.float32) for _ in range(D // 16)]
            trow = title_v[r, :]
            cnt = plsc.all_reduce_population_count(trow != 0)
            offs = trow * D
            for t in range(T):
                off = offs[t]
                for kk in range(D // 16):
                    g = table_v[pl.ds(off + kk * 16, 16)]
                    accs[kk] = accs[kk] + g
            scale = one16 / jnp.maximum(cnt.astype(jnp.float32), one16)
            for kk in range(D // 16):
                tsum_v[r, pl.ds(kk * 16, 16)] = accs[kk] * scale
            return carry

        lax.fori_loop(0, BPW, row_body, 0)
        pltpu.sync_copy(tsum_v, text2_hbm.at[pl.ds(base, BPW)])

    return k(title, text_flat)


def _sc_pk(item_id, pk_table, combo, price, ts, wmisc_flat):
    @functools.partial(
        pl.kernel,
        out_type=[
            jax.ShapeDtypeStruct((B, D), jnp.float32),
            jax.ShapeDtypeStruct((B, D), jnp.float32),
        ],
        mesh=plsc.VectorSubcoreMesh(**_MESH),
        compiler_params=_SC_PARAMS,
        scratch_types=[
            pltpu.VMEM((BPW,), jnp.int32),
            pltpu.VMEM((BPW,), jnp.int32),
            pltpu.VMEM((BPW,), jnp.float32),
            pltpu.VMEM((BPW,), jnp.float32),
            pltpu.VMEM((130 * D,), jnp.float32),
            pltpu.VMEM((PK_CH, D), jnp.float32),
            pltpu.VMEM((PK_CH, D), jnp.float32),
            pltpu.VMEM((BPW, D), jnp.float32),
            pltpu.SemaphoreType.DMA,
            pltpu.SemaphoreType.DMA,
        ],
    )
    def k(item_hbm, pk_hbm, combo_hbm, price_hbm, ts_hbm, wm_hbm,
          pk2_hbm, misc2_hbm,
          idx_v, combo_v, price_v, ts_v, wm_v, rows_a, rows_b, misc_v,
          sem_a, sem_b):
        wid = lax.axis_index("s") * NC + lax.axis_index("c")
        base = wid * BPW
        pltpu.sync_copy(item_hbm.at[pl.ds(base, BPW)], idx_v)
        bufs = [rows_a, rows_b]
        sems = [sem_a, sem_b]
        cps = [None, None]
        cps[0] = pltpu.async_copy(
            pk_hbm.at[idx_v.at[pl.ds(0, PK_CH)]], rows_a, sem_a)
        pltpu.sync_copy(combo_hbm.at[pl.ds(base, BPW)], combo_v)
        pltpu.sync_copy(price_hbm.at[pl.ds(base, BPW)], price_v)
        pltpu.sync_copy(ts_hbm.at[pl.ds(base, BPW)], ts_v)
        pltpu.sync_copy(wm_hbm, wm_v)

        wp = [wm_v[pl.ds(128 * D + kk * 16, 16)] for kk in range(D // 16)]
        wt = [wm_v[pl.ds(129 * D + kk * 16, 16)] for kk in range(D // 16)]

        def misc_body(r16, carry):
            r = r16 * 16
            cvec = combo_v[pl.ds(r, 16)] * D
            pvec = price_v[pl.ds(r, 16)]
            tvec = ts_v[pl.ds(r, 16)]
            for j in range(16):
                co = cvec[j]
                p = pvec[j]
                t = tvec[j]
                for kk in range(D // 16):
                    acc = (wm_v[pl.ds(co + kk * 16, 16)]
                           + p * wp[kk] + t * wt[kk])
                    misc_v[r + j, pl.ds(kk * 16, 16)] = acc
            return carry

        lax.fori_loop(0, BPW // 16, misc_body, 0)
        pltpu.sync_copy(misc_v, misc2_hbm.at[pl.ds(base, BPW)])

        for c in range(1, BPW // PK_CH):
            cps[c % 2] = pltpu.async_copy(
                pk_hbm.at[idx_v.at[pl.ds(c * PK_CH, PK_CH)]],
                bufs[c % 2], sems[c % 2])
        for c in range(BPW // PK_CH):
            cps[c % 2].wait()
            pltpu.sync_copy(bufs[c % 2],
                            pk2_hbm.at[pl.ds(base + c * PK_CH, PK_CH)])

    return k(item_id, pk_table, combo, price, ts, wmisc_flat)


def _tc_fnn(pk2, text2, misc2, W1pk, W1text, W2, b2):
    BLK = 1024
    grid = (HB // BLK,)
    row_spec = pl.BlockSpec((BLK, 2 * D), lambda i: (i, 0))
    full_spec = pl.BlockSpec((D, D), lambda i: (0, 0))

    def body(pk_ref, text_ref, misc_ref, w1pk_ref, w1t_ref, w2_ref, b2_ref,
             out_ref):
        halves = []
        for h in (0, D):
            x1 = jnp.dot(pk_ref[:, h:h + D], w1pk_ref[...],
                         preferred_element_type=jnp.float32)
            x1 += jnp.dot(text_ref[:, h:h + D], w1t_ref[...],
                          preferred_element_type=jnp.float32)
            x1 += misc_ref[:, h:h + D]
            hrelu = jnp.maximum(x1, 0.0)
            halves.append(jnp.dot(hrelu, w2_ref[...],
                                  preferred_element_type=jnp.float32)
                          + b2_ref[...])
        out_ref[...] = jnp.stack(halves, axis=1).reshape(2 * BLK, D)

    return pl.pallas_call(
        body,
        grid=grid,
        in_specs=[
            row_spec,
            row_spec,
            row_spec,
            full_spec,
            full_spec,
            full_spec,
            pl.BlockSpec((1, D), lambda i: (0, 0)),
        ],
        out_specs=pl.BlockSpec((2 * BLK, D), lambda i: (i, 0)),
        out_shape=jax.ShapeDtypeStruct((B, D), jnp.float32),
    )(pk2, text2, misc2, W1pk, W1text, W2, b2)


def kernel(item_id, category, brand, title, price, created_at,
           pk_table, text_table, W1, b1, W2, b2):
    item_id = item_id.astype(jnp.int32)
    title_i = title.astype(jnp.int32)
    text2 = _sc_pool(title_i, text_table.reshape(-1))
    combo_tab = (W1[D:D + 16][:, None, :] + W1[D + 16:D + 24][None, :, :]
                 + b1[None, None, :]).reshape(128, D)
    wmisc = jnp.concatenate([combo_tab, W1[D + 24 + D:]], axis=0)
    combo = category.astype(jnp.int32) * 8 + brand.astype(jnp.int32)
    pk_rows, misc_rows = _sc_pk(item_id, pk_table, combo,
                                price, created_at, wmisc.reshape(-1))
    pk2 = pk_rows.reshape(HB, 2 * D)
    misc2 = misc_rows.reshape(HB, 2 * D)
    return _tc_fnn(pk2, text2.reshape(HB, 2 * D), misc2,
                   W1[0:D], W1[D + 24:D + 24 + D], W2, b2.reshape(1, D))


# speedup vs baseline: 1.7183x; 1.0050x over previous
import functools

import jax
import jax.numpy as jnp
from jax import lax
from jax.experimental import pallas as pl
from jax.experimental.pallas import tpu as pltpu
from jax.experimental.pallas import tpu_sc as plsc

B = 16384
HB = B // 2
PK = 100001
D = 64
TV = 1000
T = 16

NC = 2
NS = 16
NW = NC * NS
BPW = B // NW
PK_CH = 256

_MESH = dict(core_axis_name="c", subcore_axis_name="s",
             num_cores=NC, num_subcores=NS)
_SC_PARAMS = pltpu.CompilerParams(needs_layout_passes=False,
                                  use_tc_tiling_on_sc=False)


def _sc_pool(title, text_flat):
    @functools.partial(
        pl.kernel,
        out_type=jax.ShapeDtypeStruct((B, D), jnp.float32),
        mesh=plsc.VectorSubcoreMesh(**_MESH),
        compiler_params=_SC_PARAMS,
        scratch_types=[
            pltpu.VMEM((BPW, T), jnp.int32),
            pltpu.VMEM((TV * D,), jnp.float32),
            pltpu.VMEM((BPW, D), jnp.float32),
        ],
    )
    def k(title_hbm, text_hbm, text2_hbm, title_v, table_v, tsum_v):
        wid = lax.axis_index("s") * NC + lax.axis_index("c")
        base = wid * BPW

        pltpu.sync_copy(text_hbm, table_v)
        pltpu.sync_copy(title_hbm.at[pl.ds(base, BPW)], title_v)

        zero16 = jnp.zeros((16,), jnp.float32)
        for kk in range(D // 16):
            table_v[pl.ds(kk * 16, 16)] = zero16

        one16 = jnp.full((16,), 1.0, jnp.float32)

        def row_body(r, carry):
            accs = [jnp.zeros((16,), jnp.float32) for _ in range(D // 16)]
            trow = title_v[r, :]
            cnt = plsc.all_reduce_population_count(trow != 0)
            offs = trow * D
            for t in range(T):
                off = offs[t]
                for kk in range(D // 16):
                    g = table_v[pl.ds(off + kk * 16, 16)]
                    accs[kk] = accs[kk] + g
            scale = one16 / jnp.maximum(cnt.astype(jnp.float32), one16)
            for kk in range(D // 16):
                tsum_v[r, pl.ds(kk * 16, 16)] = accs[kk] * scale
            return carry

        lax.fori_loop(0, BPW, row_body, 0)
        pltpu.sync_copy(tsum_v, text2_hbm.at[pl.ds(base, BPW)])

    return k(title, text_flat)


def _sc_pk(item_id, pk_table, combo, price, ts, wmisc_flat):
    @functools.partial(
        pl.kernel,
        out_type=[
            jax.ShapeDtypeStruct((B, D), jnp.float32),
            jax.ShapeDtypeStruct((B, D), jnp.float32),
        ],
        mesh=plsc.VectorSubcoreMesh(**_MESH),
        compiler_params=_SC_PARAMS,
        scratch_types=[
            pltpu.VMEM((BPW,), jnp.int32),
            pltpu.VMEM((BPW,), jnp.int32),
            pltpu.VMEM((BPW,), jnp.float32),
            pltpu.VMEM((BPW,), jnp.float32),
            pltpu.VMEM((130 * D,), jnp.float32),
            pltpu.VMEM((PK_CH, D), jnp.float32),
            pltpu.VMEM((PK_CH, D), jnp.float32),
            pltpu.VMEM((BPW, D), jnp.float32),
            pltpu.SemaphoreType.DMA,
            pltpu.SemaphoreType.DMA,
        ],
    )
    def k(item_hbm, pk_hbm, combo_hbm, price_hbm, ts_hbm, wm_hbm,
          pk2_hbm, misc2_hbm,
          idx_v, combo_v, price_v, ts_v, wm_v, rows_a, rows_b, misc_v,
          sem_a, sem_b):
        wid = lax.axis_index("s") * NC + lax.axis_index("c")
        base = wid * BPW
        pltpu.sync_copy(item_hbm.at[pl.ds(base, BPW)], idx_v)
        bufs = [rows_a, rows_b]
        sems = [sem_a, sem_b]
        cps = [None, None]
        cps[0] = pltpu.async_copy(
            pk_hbm.at[idx_v.at[pl.ds(0, PK_CH)]], rows_a, sem_a)
        pltpu.sync_copy(combo_hbm.at[pl.ds(base, BPW)], combo_v)
        pltpu.sync_copy(price_hbm.at[pl.ds(base, BPW)], price_v)
        pltpu.sync_copy(ts_hbm.at[pl.ds(base, BPW)], ts_v)
        pltpu.sync_copy(wm_hbm, wm_v)

        wp = [wm_v[pl.ds(128 * D + kk * 16, 16)] for kk in range(D // 16)]
        wt = [wm_v[pl.ds(129 * D + kk * 16, 16)] for kk in range(D // 16)]

        def misc_body(r16, carry):
            r = r16 * 16
            cvec = combo_v[pl.ds(r, 16)] * D
            pvec = price_v[pl.ds(r, 16)]
            tvec = ts_v[pl.ds(r, 16)]
            for j in range(16):
                co = cvec[j]
                p = pvec[j]
                t = tvec[j]
                for kk in range(D // 16):
                    acc = (wm_v[pl.ds(co + kk * 16, 16)]
                           + p * wp[kk] + t * wt[kk])
                    misc_v[r + j, pl.ds(kk * 16, 16)] = acc
            return carry

        lax.fori_loop(0, BPW // 16, misc_body, 0)
        pltpu.sync_copy(misc_v, misc2_hbm.at[pl.ds(base, BPW)])

        for c in range(1, BPW // PK_CH):
            cps[c % 2] = pltpu.async_copy(
                pk_hbm.at[idx_v.at[pl.ds(c * PK_CH, PK_CH)]],
                bufs[c % 2], sems[c % 2])
        for c in range(BPW // PK_CH):
            cps[c % 2].wait()
            pltpu.sync_copy(bufs[c % 2],
                            pk2_hbm.at[pl.ds(base + c * PK_CH, PK_CH)])

    return k(item_id, pk_table, combo, price, ts, wmisc_flat)


def _tc_fnn(pk2, text2, misc2, W1pk, W1text, W2, b2):
    BLK = 2048
    grid = (HB // BLK,)
    row_spec = pl.BlockSpec((BLK, 2 * D), lambda i: (i, 0))
    full_spec = pl.BlockSpec((D, D), lambda i: (0, 0))

    def body(pk_ref, text_ref, misc_ref, w1pk_ref, w1t_ref, w2_ref, b2_ref,
             out_ref):
        halves = []
        for h in (0, D):
            x1 = jnp.dot(pk_ref[:, h:h + D], w1pk_ref[...],
                         preferred_element_type=jnp.float32)
            x1 += jnp.dot(text_ref[:, h:h + D], w1t_ref[...],
                          preferred_element_type=jnp.float32)
            x1 += misc_ref[:, h:h + D]
            hrelu = jnp.maximum(x1, 0.0)
            halves.append(jnp.dot(hrelu, w2_ref[...],
                                  preferred_element_type=jnp.float32)
                          + b2_ref[...])
        out_ref[...] = jnp.stack(halves, axis=1).reshape(2 * BLK, D)

    return pl.pallas_call(
        body,
        grid=grid,
        in_specs=[
            row_spec,
            row_spec,
            row_spec,
            full_spec,
            full_spec,
            full_spec,
            pl.BlockSpec((1, D), lambda i: (0, 0)),
        ],
        out_specs=pl.BlockSpec((2 * BLK, D), lambda i: (i, 0)),
        out_shape=jax.ShapeDtypeStruct((B, D), jnp.float32),
    )(pk2, text2, misc2, W1pk, W1text, W2, b2)


def kernel(item_id, category, brand, title, price, created_at,
           pk_table, text_table, W1, b1, W2, b2):
    item_id = item_id.astype(jnp.int32)
    title_i = title.astype(jnp.int32)
    text2 = _sc_pool(title_i, text_table.reshape(-1))
    combo_tab = (W1[D:D + 16][:, None, :] + W1[D + 16:D + 24][None, :, :]
                 + b1[None, None, :]).reshape(128, D)
    wmisc = jnp.concatenate([combo_tab, W1[D + 24 + D:]], axis=0)
    combo = category.astype(jnp.int32) * 8 + brand.astype(jnp.int32)
    pk_rows, misc_rows = _sc_pk(item_id, pk_table, combo,
                                price, created_at, wmisc.reshape(-1))
    pk2 = pk_rows.reshape(HB, 2 * D)
    misc2 = misc_rows.reshape(HB, 2 * D)
    return _tc_fnn(pk2, text2.reshape(HB, 2 * D), misc2,
                   W1[0:D], W1[D + 24:D + 24 + D], W2, b2.reshape(1, D))
